# baseline (speedup 1.0000x reference)
import math

import jax
import jax.numpy as jnp
from jax.experimental import pallas as pl
from jax.experimental.pallas import tpu as pltpu

_B, _Z, _D, _OUT = 64, 1024, 1024, 256
_NBUF = 4


def _zoner_body(txt_ref, zone_hbm, wt_ref, bt_ref, wz_ref, bz_ref,
                out_ref, t_ref, zbuf, sems):
    b = pl.program_id(0)

    def _copies(row):
        slot = jax.lax.rem(row, _NBUF)
        return [
            pltpu.make_async_copy(
                zone_hbm.at[pl.ds(row, 1), pl.ds(h * (_Z // 4), _Z // 4)],
                zbuf.at[pl.ds(slot, 1), pl.ds(h * (_Z // 4), _Z // 4)],
                sems.at[slot])
            for h in range(4)
        ]

    def _start(row):
        for c in _copies(row):
            c.start()

    def _wait(row):
        for c in _copies(row):
            c.wait()

    @pl.when(b == 0)
    def _():
        for k in range(_NBUF - 1):
            _start(k)
        t_ref[...] = jnp.tanh(
            jnp.dot(txt_ref[...].astype(jnp.bfloat16), wt_ref[...],
                    preferred_element_type=jnp.float32)
            + bt_ref[...]) * (1.0 / math.sqrt(_D))

    @pl.when(b + _NBUF - 1 < _B)
    def _():
        _start(b + _NBUF - 1)

    _wait(b)
    zrow = zbuf[pl.ds(jax.lax.rem(b, _NBUF), 1)][0]
    tb = t_ref[pl.ds(b, 1), :].astype(jnp.bfloat16)
    z = jnp.tanh(
        jnp.dot(zrow, wz_ref[...],
                preferred_element_type=jnp.float32) + bz_ref[...])
    s = jax.lax.dot_general(tb, z.astype(jnp.bfloat16),
                            (((1,), (1,)), ((), ())),
                            preferred_element_type=jnp.float32)
    e = jnp.exp(s)
    out_ref[0] = e / jnp.sum(e, axis=1, keepdims=True)


def kernel(txt_embeds, zone_embeds, W_txt, b_txt, W_zone, b_zone):
    bt = b_txt.reshape(1, _OUT)
    bz = b_zone.reshape(1, _OUT)
    return pl.pallas_call(
        _zoner_body,
        grid=(_B,),
        in_specs=[
            pl.BlockSpec((_B, _D), lambda b: (0, 0)),
            pl.BlockSpec(memory_space=pltpu.MemorySpace.HBM),
            pl.BlockSpec((_D, _OUT), lambda b: (0, 0)),
            pl.BlockSpec((1, _OUT), lambda b: (0, 0)),
            pl.BlockSpec((_D, _OUT), lambda b: (0, 0)),
            pl.BlockSpec((1, _OUT), lambda b: (0, 0)),
        ],
        out_specs=pl.BlockSpec((1, 1, _Z), lambda b: (b, 0, 0)),
        out_shape=jax.ShapeDtypeStruct((_B, 1, _Z), jnp.float32),
        scratch_shapes=[
            pltpu.VMEM((_B, _OUT), jnp.float32),
            pltpu.VMEM((_NBUF, _Z, _D), jnp.float32),
            pltpu.SemaphoreType.DMA((_NBUF,)),
        ],
    )(txt_embeds, zone_embeds,
      W_txt.astype(jnp.bfloat16), bt,
      W_zone, bz).reshape(_B, _Z)


# speedup vs baseline: 1.0198x; 1.0198x over previous
import math

import jax
import jax.numpy as jnp
from jax.experimental import pallas as pl
from jax.experimental.pallas import tpu as pltpu

_B, _Z, _D, _OUT = 64, 1024, 1024, 256
_NBUF = 4


def _zoner_body(txt_ref, zone_hbm, wt_ref, bt_ref, wz_ref, bz_ref,
                out_ref, t_ref, zbuf, sems):
    b = pl.program_id(0)

    def _copies(row):
        slot = jax.lax.rem(row, _NBUF)
        return [
            pltpu.make_async_copy(
                zone_hbm.at[pl.ds(row, 1), pl.ds(h * (_Z // 2), _Z // 2)],
                zbuf.at[pl.ds(slot, 1), pl.ds(h * (_Z // 2), _Z // 2)],
                sems.at[slot])
            for h in range(2)
        ]

    def _start(row):
        for c in _copies(row):
            c.start()

    def _wait(row):
        for c in _copies(row):
            c.wait()

    @pl.when(b == 0)
    def _():
        for k in range(_NBUF - 1):
            _start(k)
        t_ref[...] = jnp.tanh(
            jnp.dot(txt_ref[...].astype(jnp.bfloat16), wt_ref[...],
                    preferred_element_type=jnp.float32)
            + bt_ref[...]) * (1.0 / math.sqrt(_D))

    @pl.when(b + _NBUF - 1 < _B)
    def _():
        _start(b + _NBUF - 1)

    _wait(b)
    zrow = zbuf[pl.ds(jax.lax.rem(b, _NBUF), 1)][0]
    tb = t_ref[pl.ds(b, 1), :].astype(jnp.bfloat16)
    z = jnp.tanh(
        jnp.dot(zrow, wz_ref[...],
                preferred_element_type=jnp.float32) + bz_ref[...])
    s = jax.lax.dot_general(tb, z.astype(jnp.bfloat16),
                            (((1,), (1,)), ((), ())),
                            preferred_element_type=jnp.float32)
    e = jnp.exp(s)
    out_ref[0] = e / jnp.sum(e, axis=1, keepdims=True)


def kernel(txt_embeds, zone_embeds, W_txt, b_txt, W_zone, b_zone):
    bt = b_txt.reshape(1, _OUT)
    bz = b_zone.reshape(1, _OUT)
    return pl.pallas_call(
        _zoner_body,
        grid=(_B,),
        in_specs=[
            pl.BlockSpec((_B, _D), lambda b: (0, 0)),
            pl.BlockSpec(memory_space=pltpu.MemorySpace.HBM),
            pl.BlockSpec((_D, _OUT), lambda b: (0, 0)),
            pl.BlockSpec((1, _OUT), lambda b: (0, 0)),
            pl.BlockSpec((_D, _OUT), lambda b: (0, 0)),
            pl.BlockSpec((1, _OUT), lambda b: (0, 0)),
        ],
        out_specs=pl.BlockSpec((1, 1, _Z), lambda b: (b, 0, 0)),
        out_shape=jax.ShapeDtypeStruct((_B, 1, _Z), jnp.float32),
        scratch_shapes=[
            pltpu.VMEM((_B, _OUT), jnp.float32),
            pltpu.VMEM((_NBUF, _Z, _D), jnp.float32),
            pltpu.SemaphoreType.DMA((_NBUF,)),
        ],
    )(txt_embeds, zone_embeds,
      W_txt.astype(jnp.bfloat16), bt,
      W_zone, bz).reshape(_B, _Z)
